# baseline (speedup 1.0000x reference)
import functools

import jax
import jax.numpy as jnp
from jax import lax
from jax.experimental import pallas as pl
from jax.experimental.pallas import tpu as pltpu
from jax.experimental.pallas import tpu_sc as plsc

N = 10000
NP = 10240
E = 320000
C = 128
NC = 2
NS = 16
NW = NC * NS
CH = 80
NCH = 128
EPT = NCH * CH
NB = 2
NCB = 4
NG = NCH // NCB
RPT = NP // NS

_mesh = plsc.VectorSubcoreMesh(core_axis_name="c", subcore_axis_name="s")


@functools.partial(
    pl.kernel,
    out_type=jax.ShapeDtypeStruct((NC, NP), jnp.float32),
    mesh=_mesh,
    scratch_types=[
        [pltpu.VMEM((CH,), jnp.int32)] * NCB,
        [pltpu.SemaphoreType.DMA] * NCB,
        pltpu.VMEM((CH,), jnp.float32),
        pltpu.VMEM((RPT,), jnp.float32),
        pltpu.VMEM_SHARED((NP,), jnp.float32),
    ],
)
def _sc_degree(row_hbm, out_hbm, rbufs, rsems, ones_v, zero_v, deg_sh):
    cid = lax.axis_index("c")
    sid = lax.axis_index("s")
    wid = sid * NC + cid

    for k in range(RPT // 16):
        zero_v[pl.ds(k * 16, 16)] = jnp.zeros((16,), jnp.float32)
    for k in range(CH // 16):
        ones_v[pl.ds(k * 16, 16)] = jnp.ones((16,), jnp.float32)
    pltpu.sync_copy(zero_v, deg_sh.at[pl.ds(sid * RPT, RPT)])
    plsc.subcore_barrier()

    for k in range(NCB):
        pltpu.async_copy(row_hbm.at[pl.ds(wid * EPT + k * CH, CH)],
                         rbufs[k], rsems[k])

    @pl.loop(0, NCH // NCB)
    def _grp(g):
        for k in range(NCB):
            c = g * NCB + k
            pltpu.make_async_copy(
                row_hbm.at[pl.ds(wid * EPT + c * CH, CH)], rbufs[k],
                rsems[k]).wait()
            pltpu.sync_copy(ones_v, deg_sh.at[rbufs[k]], add=True)

            @pl.when(c + NCB < NCH)
            def _next():
                pltpu.async_copy(
                    row_hbm.at[pl.ds(wid * EPT + (c + NCB) * CH, CH)],
                    rbufs[k], rsems[k])

    plsc.subcore_barrier()
    pltpu.sync_copy(deg_sh.at[pl.ds(sid * RPT, RPT)],
                    out_hbm.at[cid, pl.ds(sid * RPT, RPT)])


@functools.partial(
    pl.kernel,
    out_type=jax.ShapeDtypeStruct((NC, NP, C), jnp.float32),
    mesh=_mesh,
    scratch_types=[
        [pltpu.VMEM((CH,), jnp.int32)] * NB,
        [pltpu.VMEM((CH,), jnp.int32)] * NB,
        [pltpu.VMEM((CH, C), jnp.float32)] * NB,
        [pltpu.SemaphoreType.DMA] * NB,
        pltpu.VMEM_SHARED((NP, C), jnp.float32),
    ],
)
def _sc_spmm(g_hbm, row_hbm, col_hbm, out_hbm, rbufs, cbufs, rows_v, gsems,
             agg_sh):
    cid = lax.axis_index("c")
    sid = lax.axis_index("s")
    wid = sid * NC + cid

    for i in range(CH):
        for j in range(C // 16):
            rows_v[0][i, pl.ds(j * 16, 16)] = jnp.zeros((16,), jnp.float32)
    for k in range(RPT // CH):
        pltpu.sync_copy(rows_v[0], agg_sh.at[pl.ds(sid * RPT + k * CH, CH)])
    plsc.subcore_barrier()

    for b in range(NB):
        pltpu.sync_copy(row_hbm.at[pl.ds(wid * EPT + b * CH, CH)], rbufs[b])
        pltpu.sync_copy(col_hbm.at[pl.ds(wid * EPT + b * CH, CH)], cbufs[b])
        pltpu.async_copy(g_hbm.at[rbufs[b]], rows_v[b], gsems[b])

    @pl.loop(0, NCH // NB)
    def _grp(g):
        for b in range(NB):
            c = g * NB + b
            pltpu.make_async_copy(g_hbm.at[rbufs[b]], rows_v[b],
                                  gsems[b]).wait()
            pltpu.sync_copy(rows_v[b], agg_sh.at[cbufs[b]], add=True)

            @pl.when(c + NB < NCH)
            def _next():
                base = wid * EPT + (c + NB) * CH
                pltpu.sync_copy(row_hbm.at[pl.ds(base, CH)], rbufs[b])
                pltpu.sync_copy(col_hbm.at[pl.ds(base, CH)], cbufs[b])
                pltpu.async_copy(g_hbm.at[rbufs[b]], rows_v[b], gsems[b])

    plsc.subcore_barrier()
    pltpu.sync_copy(agg_sh.at[pl.ds(sid * RPT, RPT)],
                    out_hbm.at[cid, pl.ds(sid * RPT, RPT)])


_R = 512


def _dis_block(dT_ref):
    deg = dT_ref[...][:, 0:1] + dT_ref[...][:, 1:2]
    return jnp.where(deg > 0.0, lax.rsqrt(deg), 0.0)


def _tc_scale_body(dT_ref, x_ref, out_ref):
    out_ref[...] = x_ref[...] * _dis_block(dT_ref)


def _tc_layer_body(s0_ref, s1_ref, dT_ref, w_ref, b_ref, out_ref, *,
                   sign, do_relu, scale_out):
    dis = _dis_block(dT_ref)
    t = (s0_ref[...] + s1_ref[...]) * dis
    u = lax.dot_general(t, w_ref[...], (((1,), (1,)), ((), ())),
                        preferred_element_type=jnp.float32)
    u = sign * u + b_ref[...]
    if do_relu:
        u = jnp.maximum(u, 0.0)
    out_ref[...] = u * dis if scale_out else u


_row_spec = pl.BlockSpec((_R, C), lambda i: (i, 0))
_dT_spec = pl.BlockSpec((_R, 2), lambda i: (i, 0))
_w_spec = pl.BlockSpec((C, C), lambda i: (0, 0))
_b_spec = pl.BlockSpec((1, C), lambda i: (0, 0))
_out_sds = jax.ShapeDtypeStruct((NP, C), jnp.float32)


def _tc_scale(dT, xp):
    return pl.pallas_call(
        _tc_scale_body,
        grid=(NP // _R,),
        in_specs=[_dT_spec, _row_spec],
        out_specs=_row_spec,
        out_shape=_out_sds,
    )(dT, xp)


def _tc_layer(s0, s1, dT, W, b2, sign, do_relu, scale_out):
    body = functools.partial(_tc_layer_body, sign=sign, do_relu=do_relu,
                             scale_out=scale_out)
    return pl.pallas_call(
        body,
        grid=(NP // _R,),
        in_specs=[_row_spec, _row_spec, _dT_spec, _w_spec, _b_spec],
        out_specs=_row_spec,
        out_shape=_out_sds,
    )(s0, s1, dT, W, b2)


def _padded(idx):
    idx = idx.reshape(NW, E // NW)
    idx = jnp.pad(idx, ((0, 0), (0, EPT - E // NW)), constant_values=NP - 1)
    return idx.reshape(NW * EPT)


def kernel(x, edge_index, W0, b0, W1, b1, W2, b2):
    xp = jnp.pad(x, ((0, NP - N), (0, 0)))
    row = _padded(edge_index[0])
    col = _padded(edge_index[1])

    degp = _sc_degree(row)
    dT = degp.T

    g = _tc_scale(dT, xp)
    params = [(W0, b0, 1.0, True, True),
              (W1, b1, -1.0, True, True),
              (W2, b2, 1.0, False, False)]
    h = None
    for W, b, sign, do_relu, scale_out in params:
        s = _sc_spmm(g, row, col)
        h = _tc_layer(s[0], s[1], dT, W, jnp.reshape(b, (1, C)),
                      sign, do_relu, scale_out)
        g = h
    return h[:N]


# speedup vs baseline: 2.2007x; 2.2007x over previous
import functools

import jax
import jax.numpy as jnp
from jax import lax
from jax.experimental import pallas as pl
from jax.experimental.pallas import tpu as pltpu
from jax.experimental.pallas import tpu_sc as plsc

N = 10000
NP = 10240
E = 320000
C = 128
NC = 2
NS = 16
NW = NC * NS
CH = 80
NCH = 128
EPT = NCH * CH
NB = 2
NCB = 4
NG = NCH // NCB
RPT = NP // NS

_mesh = plsc.VectorSubcoreMesh(core_axis_name="c", subcore_axis_name="s")


@functools.partial(
    pl.kernel,
    out_type=jax.ShapeDtypeStruct((NC, NP), jnp.float32),
    mesh=_mesh,
    scratch_types=[
        [pltpu.VMEM((CH,), jnp.int32)] * NCB,
        [pltpu.SemaphoreType.DMA] * NCB,
        pltpu.VMEM((CH,), jnp.float32),
        pltpu.VMEM((RPT,), jnp.float32),
        pltpu.VMEM_SHARED((NP,), jnp.float32),
    ],
)
def _sc_degree(row_hbm, out_hbm, rbufs, rsems, ones_v, zero_v, deg_sh):
    cid = lax.axis_index("c")
    sid = lax.axis_index("s")
    wid = sid * NC + cid

    for k in range(RPT // 16):
        zero_v[pl.ds(k * 16, 16)] = jnp.zeros((16,), jnp.float32)
    for k in range(CH // 16):
        ones_v[pl.ds(k * 16, 16)] = jnp.ones((16,), jnp.float32)
    pltpu.sync_copy(zero_v, deg_sh.at[pl.ds(sid * RPT, RPT)])
    plsc.subcore_barrier()

    for k in range(NCB):
        pltpu.async_copy(row_hbm.at[pl.ds(wid * EPT + k * CH, CH)],
                         rbufs[k], rsems[k])

    @pl.loop(0, NCH // NCB)
    def _grp(g):
        for k in range(NCB):
            c = g * NCB + k
            pltpu.make_async_copy(
                row_hbm.at[pl.ds(wid * EPT + c * CH, CH)], rbufs[k],
                rsems[k]).wait()
            pltpu.sync_copy(ones_v, deg_sh.at[rbufs[k]], add=True)

            @pl.when(c + NCB < NCH)
            def _next():
                pltpu.async_copy(
                    row_hbm.at[pl.ds(wid * EPT + (c + NCB) * CH, CH)],
                    rbufs[k], rsems[k])

    plsc.subcore_barrier()
    pltpu.sync_copy(deg_sh.at[pl.ds(sid * RPT, RPT)],
                    out_hbm.at[cid, pl.ds(sid * RPT, RPT)])


@functools.partial(
    pl.kernel,
    out_type=jax.ShapeDtypeStruct((NC, NP, C), jnp.float32),
    mesh=_mesh,
    scratch_types=[
        [pltpu.VMEM((CH,), jnp.int32)] * NB,
        [pltpu.VMEM((CH,), jnp.int32)] * NB,
        [pltpu.VMEM((CH, C), jnp.float32)] * NB,
        [pltpu.SemaphoreType.DMA] * NB,
        pltpu.VMEM_SHARED((NP, C), jnp.float32),
    ],
)
def _sc_spmm(g_hbm, row_hbm, col_hbm, out_hbm, rbufs, cbufs, rows_v, gsems,
             agg_sh):
    cid = lax.axis_index("c")
    sid = lax.axis_index("s")
    wid = sid * NC + cid

    for i in range(CH):
        for j in range(C // 16):
            rows_v[0][i, pl.ds(j * 16, 16)] = jnp.zeros((16,), jnp.float32)
    for k in range(RPT // CH):
        pltpu.sync_copy(rows_v[0], agg_sh.at[pl.ds(sid * RPT + k * CH, CH)])
    plsc.subcore_barrier()

    for b in range(NB):
        pltpu.sync_copy(row_hbm.at[pl.ds(wid * EPT + b * CH, CH)], rbufs[b])
        pltpu.sync_copy(col_hbm.at[pl.ds(wid * EPT + b * CH, CH)], cbufs[b])
        pltpu.async_copy(g_hbm.at[rbufs[b]], rows_v[b], gsems[b])

    @pl.loop(0, NCH // NB)
    def _grp(g):
        for b in range(NB):
            c = g * NB + b
            pltpu.make_async_copy(g_hbm.at[rbufs[b]], rows_v[b],
                                  gsems[b]).wait()
            pltpu.sync_copy(rows_v[b], agg_sh.at[cbufs[b]], add=True)

            @pl.when(c + NB < NCH)
            def _next():
                base = wid * EPT + (c + NB) * CH
                pltpu.sync_copy(row_hbm.at[pl.ds(base, CH)], rbufs[b])
                pltpu.sync_copy(col_hbm.at[pl.ds(base, CH)], cbufs[b])
                pltpu.async_copy(g_hbm.at[rbufs[b]], rows_v[b], gsems[b])

    plsc.subcore_barrier()
    pltpu.sync_copy(agg_sh.at[pl.ds(sid * RPT, RPT)],
                    out_hbm.at[cid, pl.ds(sid * RPT, RPT)])


_R = 512


def _dis_block(dT_ref):
    deg = dT_ref[...][:, 0:1] + dT_ref[...][:, 1:2]
    return jnp.where(deg > 0.0, lax.rsqrt(deg), 0.0)


def _tc_scale_body(dT_ref, x_ref, out_ref):
    out_ref[...] = x_ref[...] * _dis_block(dT_ref)


def _tc_layer_body(s0_ref, s1_ref, dT_ref, w_ref, b_ref, out_ref, *,
                   sign, do_relu, scale_out):
    dis = _dis_block(dT_ref)
    t = (s0_ref[...] + s1_ref[...]) * dis
    u = lax.dot_general(t, w_ref[...], (((1,), (1,)), ((), ())),
                        preferred_element_type=jnp.float32)
    u = sign * u + b_ref[...]
    if do_relu:
        u = jnp.maximum(u, 0.0)
    out_ref[...] = u * dis if scale_out else u


_row_spec = pl.BlockSpec((_R, C), lambda i: (i, 0))
_dT_spec = pl.BlockSpec((_R, 2), lambda i: (i, 0))
_w_spec = pl.BlockSpec((C, C), lambda i: (0, 0))
_b_spec = pl.BlockSpec((1, C), lambda i: (0, 0))
_out_sds = jax.ShapeDtypeStruct((NP, C), jnp.float32)


def _tc_scale(dT, xp):
    return pl.pallas_call(
        _tc_scale_body,
        grid=(NP // _R,),
        in_specs=[_dT_spec, _row_spec],
        out_specs=_row_spec,
        out_shape=_out_sds,
    )(dT, xp)


def _tc_layer(s0, s1, dT, W, b2, sign, do_relu, scale_out):
    body = functools.partial(_tc_layer_body, sign=sign, do_relu=do_relu,
                             scale_out=scale_out)
    return pl.pallas_call(
        body,
        grid=(NP // _R,),
        in_specs=[_row_spec, _row_spec, _dT_spec, _w_spec, _b_spec],
        out_specs=_row_spec,
        out_shape=_out_sds,
    )(s0, s1, dT, W, b2)


def _padded(idx):
    idx = idx.reshape(NW, E // NW)
    pad = (jnp.arange(EPT - E // NW, dtype=jnp.int32)[None, :]
           + 8 * jnp.arange(NW, dtype=jnp.int32)[:, None]) % (NP - N) + N
    return jnp.concatenate([idx, pad], axis=1).reshape(NW * EPT)


def kernel(x, edge_index, W0, b0, W1, b1, W2, b2):
    xp = jnp.pad(x, ((0, NP - N), (0, 0)))
    row = _padded(edge_index[0])
    col = _padded(edge_index[1])

    degp = _sc_degree(row)
    dT = degp.T

    g = _tc_scale(dT, xp)
    params = [(W0, b0, 1.0, True, True),
              (W1, b1, -1.0, True, True),
              (W2, b2, 1.0, False, False)]
    h = None
    for W, b, sign, do_relu, scale_out in params:
        s = _sc_spmm(g, row, col)
        h = _tc_layer(s[0], s[1], dT, W, jnp.reshape(b, (1, C)),
                      sign, do_relu, scale_out)
        g = h
    return h[:N]


# speedup vs baseline: 3.0672x; 1.3937x over previous
import functools

import jax
import jax.numpy as jnp
from jax import lax
from jax.experimental import pallas as pl
from jax.experimental.pallas import tpu as pltpu
from jax.experimental.pallas import tpu_sc as plsc

N = 10000
NP = 10240
E = 320000
C = 128
NC = 2
NS = 16
NW = NC * NS
CH = 80
NCH = 128
EPT = NCH * CH
NB = 2
NCB = 4
NG = NCH // NCB
RPT = NP // NS

_mesh = plsc.VectorSubcoreMesh(core_axis_name="c", subcore_axis_name="s")


@functools.partial(
    pl.kernel,
    out_type=jax.ShapeDtypeStruct((NC, NP), jnp.float32),
    mesh=_mesh,
    scratch_types=[
        [pltpu.VMEM((CH,), jnp.int32)] * NCB,
        [pltpu.SemaphoreType.DMA] * NCB,
        pltpu.VMEM((CH,), jnp.float32),
        pltpu.VMEM((RPT,), jnp.float32),
        pltpu.VMEM_SHARED((NP,), jnp.float32),
    ],
)
def _sc_degree(row_hbm, out_hbm, rbufs, rsems, ones_v, zero_v, deg_sh):
    cid = lax.axis_index("c")
    sid = lax.axis_index("s")
    wid = sid * NC + cid

    for k in range(RPT // 16):
        zero_v[pl.ds(k * 16, 16)] = jnp.zeros((16,), jnp.float32)
    for k in range(CH // 16):
        ones_v[pl.ds(k * 16, 16)] = jnp.ones((16,), jnp.float32)
    pltpu.sync_copy(zero_v, deg_sh.at[pl.ds(sid * RPT, RPT)])
    plsc.subcore_barrier()

    for k in range(NCB):
        pltpu.async_copy(row_hbm.at[pl.ds(wid * EPT + k * CH, CH)],
                         rbufs[k], rsems[k])

    @pl.loop(0, NCH // NCB)
    def _grp(g):
        for k in range(NCB):
            c = g * NCB + k
            pltpu.make_async_copy(
                row_hbm.at[pl.ds(wid * EPT + c * CH, CH)], rbufs[k],
                rsems[k]).wait()
            pltpu.sync_copy(ones_v, deg_sh.at[rbufs[k]], add=True)

            @pl.when(c + NCB < NCH)
            def _next():
                pltpu.async_copy(
                    row_hbm.at[pl.ds(wid * EPT + (c + NCB) * CH, CH)],
                    rbufs[k], rsems[k])

    plsc.subcore_barrier()
    pltpu.sync_copy(deg_sh.at[pl.ds(sid * RPT, RPT)],
                    out_hbm.at[cid, pl.ds(sid * RPT, RPT)])


@functools.partial(
    pl.kernel,
    out_type=jax.ShapeDtypeStruct((NC, NP, C), jnp.float32),
    mesh=_mesh,
    scratch_types=[
        [pltpu.VMEM((CH,), jnp.int32)] * NCB,
        [pltpu.VMEM((CH,), jnp.int32)] * NCB,
        [pltpu.VMEM((CH, C), jnp.float32)] * NB,
        [pltpu.SemaphoreType.DMA] * NB,
        [pltpu.SemaphoreType.DMA] * NCB,
        [pltpu.SemaphoreType.DMA] * NCB,
        pltpu.VMEM_SHARED((NP, C), jnp.float32),
    ],
)
def _sc_spmm(g_hbm, row_hbm, col_hbm, out_hbm, rbufs, cbufs, rows_v, gsems,
             rsems, csems, agg_sh):
    cid = lax.axis_index("c")
    sid = lax.axis_index("s")
    wid = sid * NC + cid

    for i in range(CH):
        for j in range(C // 16):
            rows_v[0][i, pl.ds(j * 16, 16)] = jnp.zeros((16,), jnp.float32)
    for k in range(RPT // CH):
        pltpu.sync_copy(rows_v[0], agg_sh.at[pl.ds(sid * RPT + k * CH, CH)])
    plsc.subcore_barrier()

    for k in range(NCB):
        pltpu.async_copy(row_hbm.at[pl.ds(wid * EPT + k * CH, CH)],
                         rbufs[k], rsems[k])
        pltpu.async_copy(col_hbm.at[pl.ds(wid * EPT + k * CH, CH)],
                         cbufs[k], csems[k])
    for b in range(NB):
        pltpu.make_async_copy(
            row_hbm.at[pl.ds(wid * EPT + b * CH, CH)], rbufs[b],
            rsems[b]).wait()
        pltpu.async_copy(g_hbm.at[rbufs[b]], rows_v[b], gsems[b])

    @pl.loop(0, NCH // NCB)
    def _grp(g):
        for k in range(NCB):
            c = g * NCB + k
            b = k % NB
            k2 = (k + NB) % NCB
            pltpu.make_async_copy(g_hbm.at[rbufs[k]], rows_v[b],
                                  gsems[b]).wait()
            pltpu.make_async_copy(
                col_hbm.at[pl.ds(wid * EPT + c * CH, CH)], cbufs[k],
                csems[k]).wait()
            pltpu.sync_copy(rows_v[b], agg_sh.at[cbufs[k]], add=True)

            @pl.when(c + NCB < NCH)
            def _nexti():
                base = wid * EPT + (c + NCB) * CH
                pltpu.async_copy(row_hbm.at[pl.ds(base, CH)], rbufs[k],
                                 rsems[k])
                pltpu.async_copy(col_hbm.at[pl.ds(base, CH)], cbufs[k],
                                 csems[k])

            @pl.when(c + NB < NCH)
            def _nextg():
                pltpu.make_async_copy(
                    row_hbm.at[pl.ds(wid * EPT + (c + NB) * CH, CH)],
                    rbufs[k2], rsems[k2]).wait()
                pltpu.async_copy(g_hbm.at[rbufs[k2]], rows_v[b], gsems[b])

    plsc.subcore_barrier()
    pltpu.sync_copy(agg_sh.at[pl.ds(sid * RPT, RPT)],
                    out_hbm.at[cid, pl.ds(sid * RPT, RPT)])


_R = 512


def _dis_block(dT_ref):
    deg = dT_ref[...][:, 0:1] + dT_ref[...][:, 1:2]
    return jnp.where(deg > 0.0, lax.rsqrt(deg), 0.0)


def _tc_scale_body(dT_ref, x_ref, out_ref):
    out_ref[...] = x_ref[...] * _dis_block(dT_ref)


def _tc_layer_body(s0_ref, s1_ref, dT_ref, w_ref, b_ref, out_ref, *,
                   sign, do_relu, scale_out):
    dis = _dis_block(dT_ref)
    t = (s0_ref[...] + s1_ref[...]) * dis
    u = lax.dot_general(t, w_ref[...], (((1,), (1,)), ((), ())),
                        preferred_element_type=jnp.float32)
    u = sign * u + b_ref[...]
    if do_relu:
        u = jnp.maximum(u, 0.0)
    out_ref[...] = u * dis if scale_out else u


_row_spec = pl.BlockSpec((_R, C), lambda i: (i, 0))
_dT_spec = pl.BlockSpec((_R, 2), lambda i: (i, 0))
_w_spec = pl.BlockSpec((C, C), lambda i: (0, 0))
_b_spec = pl.BlockSpec((1, C), lambda i: (0, 0))
_out_sds = jax.ShapeDtypeStruct((NP, C), jnp.float32)


def _tc_scale(dT, xp):
    return pl.pallas_call(
        _tc_scale_body,
        grid=(NP // _R,),
        in_specs=[_dT_spec, _row_spec],
        out_specs=_row_spec,
        out_shape=_out_sds,
    )(dT, xp)


def _tc_layer(s0, s1, dT, W, b2, sign, do_relu, scale_out):
    body = functools.partial(_tc_layer_body, sign=sign, do_relu=do_relu,
                             scale_out=scale_out)
    return pl.pallas_call(
        body,
        grid=(NP // _R,),
        in_specs=[_row_spec, _row_spec, _dT_spec, _w_spec, _b_spec],
        out_specs=_row_spec,
        out_shape=_out_sds,
    )(s0, s1, dT, W, b2)


def _padded(idx):
    idx = idx.reshape(NW, E // NW)
    pad = (jnp.arange(EPT - E // NW, dtype=jnp.int32)[None, :]
           + 8 * jnp.arange(NW, dtype=jnp.int32)[:, None]) % (NP - N) + N
    return jnp.concatenate([idx, pad], axis=1).reshape(NW * EPT)


def kernel(x, edge_index, W0, b0, W1, b1, W2, b2):
    xp = jnp.pad(x, ((0, NP - N), (0, 0)))
    row = _padded(edge_index[0])
    col = _padded(edge_index[1])

    degp = _sc_degree(row)
    dT = degp.T

    g = _tc_scale(dT, xp)
    params = [(W0, b0, 1.0, True, True),
              (W1, b1, -1.0, True, True),
              (W2, b2, 1.0, False, False)]
    h = None
    for W, b, sign, do_relu, scale_out in params:
        s = _sc_spmm(g, row, col)
        h = _tc_layer(s[0], s[1], dT, W, jnp.reshape(b, (1, C)),
                      sign, do_relu, scale_out)
        g = h
    return h[:N]


# speedup vs baseline: 3.3763x; 1.1008x over previous
import functools

import jax
import jax.numpy as jnp
from jax import lax
from jax.experimental import pallas as pl
from jax.experimental.pallas import tpu as pltpu
from jax.experimental.pallas import tpu_sc as plsc

N = 10000
NP = 10240
E = 320000
C = 128
NC = 2
NS = 16
NW = NC * NS
CH = 80
NCH = 132
EPT = NCH * CH
NB = 3
NCB = 6
NG = NCH // NCB
RPT = NP // NS

_mesh = plsc.VectorSubcoreMesh(core_axis_name="c", subcore_axis_name="s")


@functools.partial(
    pl.kernel,
    out_type=jax.ShapeDtypeStruct((NC, NP), jnp.float32),
    mesh=_mesh,
    scratch_types=[
        [pltpu.VMEM((CH,), jnp.int32)] * 4,
        [pltpu.SemaphoreType.DMA] * 4,
        pltpu.VMEM((CH,), jnp.float32),
        pltpu.VMEM((RPT,), jnp.float32),
        pltpu.VMEM_SHARED((NP,), jnp.float32),
    ],
)
def _sc_degree(row_hbm, out_hbm, rbufs, rsems, ones_v, zero_v, deg_sh):
    cid = lax.axis_index("c")
    sid = lax.axis_index("s")
    wid = sid * NC + cid

    for k in range(RPT // 16):
        zero_v[pl.ds(k * 16, 16)] = jnp.zeros((16,), jnp.float32)
    for k in range(CH // 16):
        ones_v[pl.ds(k * 16, 16)] = jnp.ones((16,), jnp.float32)
    pltpu.sync_copy(zero_v, deg_sh.at[pl.ds(sid * RPT, RPT)])
    plsc.subcore_barrier()

    for k in range(4):
        pltpu.async_copy(row_hbm.at[pl.ds(wid * EPT + k * CH, CH)],
                         rbufs[k], rsems[k])

    @pl.loop(0, NCH // 4)
    def _grp(g):
        for k in range(4):
            c = g * 4 + k
            pltpu.make_async_copy(
                row_hbm.at[pl.ds(wid * EPT + c * CH, CH)], rbufs[k],
                rsems[k]).wait()
            pltpu.sync_copy(ones_v, deg_sh.at[rbufs[k]], add=True)

            @pl.when(c + 4 < NCH)
            def _next():
                pltpu.async_copy(
                    row_hbm.at[pl.ds(wid * EPT + (c + 4) * CH, CH)],
                    rbufs[k], rsems[k])

    plsc.subcore_barrier()
    pltpu.sync_copy(deg_sh.at[pl.ds(sid * RPT, RPT)],
                    out_hbm.at[cid, pl.ds(sid * RPT, RPT)])


@functools.partial(
    pl.kernel,
    out_type=jax.ShapeDtypeStruct((NC, NP, C), jnp.float32),
    mesh=_mesh,
    scratch_types=[
        [pltpu.VMEM((CH,), jnp.int32)] * NCB,
        [pltpu.VMEM((CH,), jnp.int32)] * NCB,
        [pltpu.VMEM((CH, C), jnp.float32)] * NB,
        [pltpu.SemaphoreType.DMA] * NB,
        [pltpu.SemaphoreType.DMA] * NCB,
        [pltpu.SemaphoreType.DMA] * NCB,
        pltpu.VMEM_SHARED((NP, C), jnp.float32),
    ],
)
def _sc_spmm(g_hbm, row_hbm, col_hbm, out_hbm, rbufs, cbufs, rows_v, gsems,
             rsems, csems, agg_sh):
    cid = lax.axis_index("c")
    sid = lax.axis_index("s")
    wid = sid * NC + cid

    for i in range(CH):
        for j in range(C // 16):
            rows_v[0][i, pl.ds(j * 16, 16)] = jnp.zeros((16,), jnp.float32)
    for k in range(RPT // CH):
        pltpu.sync_copy(rows_v[0], agg_sh.at[pl.ds(sid * RPT + k * CH, CH)])
    plsc.subcore_barrier()

    for k in range(NCB):
        pltpu.async_copy(row_hbm.at[pl.ds(wid * EPT + k * CH, CH)],
                         rbufs[k], rsems[k])
        pltpu.async_copy(col_hbm.at[pl.ds(wid * EPT + k * CH, CH)],
                         cbufs[k], csems[k])
    for b in range(NB):
        pltpu.make_async_copy(
            row_hbm.at[pl.ds(wid * EPT + b * CH, CH)], rbufs[b],
            rsems[b]).wait()
        pltpu.async_copy(g_hbm.at[rbufs[b]], rows_v[b], gsems[b])

    @pl.loop(0, NCH // NCB)
    def _grp(g):
        for k in range(NCB):
            c = g * NCB + k
            b = k % NB
            k2 = (k + NB) % NCB
            pltpu.make_async_copy(g_hbm.at[rbufs[k]], rows_v[b],
                                  gsems[b]).wait()
            pltpu.make_async_copy(
                col_hbm.at[pl.ds(wid * EPT + c * CH, CH)], cbufs[k],
                csems[k]).wait()
            pltpu.sync_copy(rows_v[b], agg_sh.at[cbufs[k]], add=True)

            @pl.when(c + NCB < NCH)
            def _nexti():
                base = wid * EPT + (c + NCB) * CH
                pltpu.async_copy(row_hbm.at[pl.ds(base, CH)], rbufs[k],
                                 rsems[k])
                pltpu.async_copy(col_hbm.at[pl.ds(base, CH)], cbufs[k],
                                 csems[k])

            @pl.when(c + NB < NCH)
            def _nextg():
                pltpu.make_async_copy(
                    row_hbm.at[pl.ds(wid * EPT + (c + NB) * CH, CH)],
                    rbufs[k2], rsems[k2]).wait()
                pltpu.async_copy(g_hbm.at[rbufs[k2]], rows_v[b], gsems[b])

    plsc.subcore_barrier()
    pltpu.sync_copy(agg_sh.at[pl.ds(sid * RPT, RPT)],
                    out_hbm.at[cid, pl.ds(sid * RPT, RPT)])


_R = 512


def _dis_block(dT_ref):
    deg = dT_ref[...][:, 0:1] + dT_ref[...][:, 1:2]
    return jnp.where(deg > 0.0, lax.rsqrt(deg), 0.0)


def _tc_scale_body(dT_ref, x_ref, out_ref):
    out_ref[...] = x_ref[...] * _dis_block(dT_ref)


def _tc_layer_body(s0_ref, s1_ref, dT_ref, w_ref, b_ref, out_ref, *,
                   sign, do_relu, scale_out):
    dis = _dis_block(dT_ref)
    t = (s0_ref[...] + s1_ref[...]) * dis
    u = lax.dot_general(t, w_ref[...], (((1,), (1,)), ((), ())),
                        preferred_element_type=jnp.float32)
    u = sign * u + b_ref[...]
    if do_relu:
        u = jnp.maximum(u, 0.0)
    out_ref[...] = u * dis if scale_out else u


_row_spec = pl.BlockSpec((_R, C), lambda i: (i, 0))
_dT_spec = pl.BlockSpec((_R, 2), lambda i: (i, 0))
_w_spec = pl.BlockSpec((C, C), lambda i: (0, 0))
_b_spec = pl.BlockSpec((1, C), lambda i: (0, 0))
_out_sds = jax.ShapeDtypeStruct((NP, C), jnp.float32)


def _tc_scale(dT, xp):
    return pl.pallas_call(
        _tc_scale_body,
        grid=(NP // _R,),
        in_specs=[_dT_spec, _row_spec],
        out_specs=_row_spec,
        out_shape=_out_sds,
    )(dT, xp)


def _tc_layer(s0, s1, dT, W, b2, sign, do_relu, scale_out):
    body = functools.partial(_tc_layer_body, sign=sign, do_relu=do_relu,
                             scale_out=scale_out)
    return pl.pallas_call(
        body,
        grid=(NP // _R,),
        in_specs=[_row_spec, _row_spec, _dT_spec, _w_spec, _b_spec],
        out_specs=_row_spec,
        out_shape=_out_sds,
    )(s0, s1, dT, W, b2)


def _padded(idx):
    idx = idx.reshape(NW, E // NW)
    pad = (jnp.arange(EPT - E // NW, dtype=jnp.int32)[None, :]
           + 8 * jnp.arange(NW, dtype=jnp.int32)[:, None]) % (NP - N) + N
    return jnp.concatenate([idx, pad], axis=1).reshape(NW * EPT)


def kernel(x, edge_index, W0, b0, W1, b1, W2, b2):
    xp = jnp.pad(x, ((0, NP - N), (0, 0)))
    row = _padded(edge_index[0])
    col = _padded(edge_index[1])

    degp = _sc_degree(row)
    dT = degp.T

    g = _tc_scale(dT, xp)
    params = [(W0, b0, 1.0, True, True),
              (W1, b1, -1.0, True, True),
              (W2, b2, 1.0, False, False)]
    h = None
    for W, b, sign, do_relu, scale_out in params:
        s = _sc_spmm(g, row, col)
        h = _tc_layer(s[0], s[1], dT, W, jnp.reshape(b, (1, C)),
                      sign, do_relu, scale_out)
        g = h
    return h[:N]
